# baseline (speedup 1.0000x reference)
import functools

import jax
import jax.numpy as jnp
from jax import lax
from jax.experimental import pallas as pl
from jax.experimental.pallas import tpu as pltpu
from jax.experimental.pallas import tpu_sc as plsc

_NUM_CORES = 2
_NUM_SUBCORES = 16
_NW = _NUM_CORES * _NUM_SUBCORES
_L = 16
_GL = 128
_Q = 1024


@functools.partial(jax.jit, static_argnums=(2, 3, 4))
def _sc_gather_t(idx3d, t16, hist, batch, d):
    ndg = d // _L
    units_per_w = hist * ndg // _NW
    nglists = batch // _GL
    mesh = plsc.VectorSubcoreMesh(core_axis_name="c", subcore_axis_name="s")

    @functools.partial(
        pl.kernel,
        mesh=mesh,
        out_type=jax.ShapeDtypeStruct((hist, d, batch), jnp.float32),
        scratch_types=[
            pltpu.VMEM((nglists, _GL), jnp.int32),
            pltpu.VMEM((nglists, _GL), jnp.int32),
            pltpu.VMEM((_Q, _L), jnp.float32),
            pltpu.VMEM((_Q, _L), jnp.float32),
            pltpu.VMEM((_L, batch), jnp.float32),
            pltpu.SemaphoreType.DMA,
            pltpu.SemaphoreType.DMA,
        ],
        compiler_params=pltpu.CompilerParams(
            use_tc_tiling_on_sc=False, needs_layout_passes=False),
    )
    def k(idx_hbm, t16_hbm, out_hbm, idxb, glb, wa, wb, outb, gsem, ssem):
        wid = lax.axis_index("s") * _NUM_CORES + lax.axis_index("c")
        iota = lax.iota(jnp.int32, _L)

        def fire_q(q_ref, base_list):
            cps = []
            for g in range(_Q // _GL):
                cps.append(pltpu.async_copy(
                    t16_hbm.at[glb.at[base_list + g]],
                    q_ref.at[pl.ds(g * _GL, _GL), :], gsem))
            return cps

        def shuffle_q(q_ref, col0):
            def piece(pp, carry):
                base = pp * 128
                for j in range(128):
                    r = base + j
                    v = plsc.load_gather(
                        q_ref, [jnp.zeros((_L,), jnp.int32) + r, iota])
                    col = jnp.zeros((_L,), jnp.int32) + col0 + r
                    plsc.store_scatter(outb, [iota, col], v)
                return carry
            lax.fori_loop(0, _Q // 128, piece, 0)

        def unit(u, carry):
            nb = wid * units_per_w + u
            h = nb // ndg
            dg = nb % ndg
            pltpu.sync_copy(idx_hbm.at[h], idxb)

            def glist(r, carry2):
                rv = jnp.zeros((_L,), jnp.int32) + r
                for c in range(_GL // _L):
                    cv = c * _L + iota
                    v = plsc.load_gather(idxb, [rv, cv])
                    plsc.store_scatter(glb, [rv, cv], v * 4 + dg)
                return carry2
            lax.fori_loop(0, nglists, glist, 0)

            bufs = [wa, wb, wa, wb]
            cps = fire_q(bufs[0], 0)
            for q in range(1, 4):
                cps_next = fire_q(bufs[q], q * (_Q // _GL))
                for cp in cps:
                    cp.wait()
                shuffle_q(bufs[q - 1], (q - 1) * _Q)
                cps = cps_next
            for cp in cps:
                cp.wait()
            shuffle_q(bufs[3], 3 * _Q)
            pltpu.sync_copy(outb, out_hbm.at[h, pl.ds(dg * _L, _L), :])
            return carry

        lax.fori_loop(0, units_per_w, unit, 0)

    return k(idx3d, t16)


def kernel(indices, table):
    b, h = indices.shape
    v, d = table.shape
    t2 = lax.optimization_barrier(
        jnp.concatenate([table[0::2], table[1::2]], axis=1))
    t16 = t2.reshape(v * (d // _L), _L)
    idx3d = indices.astype(jnp.int32).T.reshape(h, b // _GL, _GL)
    outp = _sc_gather_t(idx3d, t16, h, b, d)
    return outp.transpose(2, 0, 1)


# speedup vs baseline: 5.7338x; 5.7338x over previous
import functools

import jax
import jax.numpy as jnp
from jax import lax
from jax.experimental import pallas as pl
from jax.experimental.pallas import tpu as pltpu
from jax.experimental.pallas import tpu_sc as plsc

_NUM_CORES = 2
_NUM_SUBCORES = 16
_NW = _NUM_CORES * _NUM_SUBCORES
_L = 16
_GL = 128
_Q = 1024


@functools.partial(jax.jit, static_argnums=(2, 3, 4))
def _sc_gather_t(idx3d, t16, hist, batch, d):
    ndg = d // _L
    units_per_w = hist * ndg // _NW
    nglists = batch // _GL
    mesh = plsc.VectorSubcoreMesh(core_axis_name="c", subcore_axis_name="s")

    @functools.partial(
        pl.kernel,
        mesh=mesh,
        out_type=jax.ShapeDtypeStruct((hist, d, batch), jnp.float32),
        scratch_types=[
            pltpu.VMEM((nglists, _GL), jnp.int32),
            pltpu.VMEM((nglists, _GL), jnp.int32),
            pltpu.VMEM((_Q, _L), jnp.float32),
            pltpu.VMEM((_Q, _L), jnp.float32),
            pltpu.VMEM((_L, batch), jnp.float32),
            pltpu.SemaphoreType.DMA,
            pltpu.SemaphoreType.DMA,
        ],
        compiler_params=pltpu.CompilerParams(
            use_tc_tiling_on_sc=False, needs_layout_passes=False),
    )
    def k(idx_hbm, t16_hbm, out_hbm, idxb, glb, wa, wb, outb, gsem, ssem):
        wid = lax.axis_index("s") * _NUM_CORES + lax.axis_index("c")
        iota = lax.iota(jnp.int32, _L)

        def fire_q(q_ref, base_list):
            cps = []
            for g in range(_Q // _GL):
                cps.append(pltpu.async_copy(
                    t16_hbm.at[glb.at[base_list + g]],
                    q_ref.at[pl.ds(g * _GL, _GL), :], gsem))
            return cps

        def shuffle_q(q_ref, col0):
            def piece(pp, carry):
                base = pp * 128
                for j0 in range(0, 128, 4):
                    vs = []
                    for j in range(j0, j0 + 4):
                        vs.append(q_ref[base + j])
                    for i, j in enumerate(range(j0, j0 + 4)):
                        col = jnp.zeros((_L,), jnp.int32) + col0 + base + j
                        plsc.store_scatter(outb, [iota, col], vs[i])
                return carry
            lax.fori_loop(0, _Q // 128, piece, 0)

        def unit(u, carry):
            nb = wid * units_per_w + u
            h = nb // ndg
            dg = nb % ndg
            pltpu.sync_copy(idx_hbm.at[h], idxb)

            def glist(r, carry2):
                rv = jnp.zeros((_L,), jnp.int32) + r
                for c in range(_GL // _L):
                    cv = c * _L + iota
                    v = plsc.load_gather(idxb, [rv, cv])
                    plsc.store_scatter(glb, [rv, cv], v * 4 + dg)
                return carry2
            lax.fori_loop(0, nglists, glist, 0)

            bufs = [wa, wb, wa, wb]
            cps = fire_q(bufs[0], 0)
            for q in range(1, 4):
                cps_next = fire_q(bufs[q], q * (_Q // _GL))
                for cp in cps:
                    cp.wait()
                shuffle_q(bufs[q - 1], (q - 1) * _Q)
                cps = cps_next
            for cp in cps:
                cp.wait()
            shuffle_q(bufs[3], 3 * _Q)
            pltpu.sync_copy(outb, out_hbm.at[h, pl.ds(dg * _L, _L), :])
            return carry

        lax.fori_loop(0, units_per_w, unit, 0)

    return k(idx3d, t16)


def kernel(indices, table):
    b, h = indices.shape
    v, d = table.shape
    t2 = lax.optimization_barrier(table.reshape(v // 2, 2 * d))
    t16 = t2.reshape(v * (d // _L), _L)
    idx3d = indices.astype(jnp.int32).T.reshape(h, b // _GL, _GL)
    outp = _sc_gather_t(idx3d, t16, h, b, d)
    return outp.transpose(2, 0, 1)


# speedup vs baseline: 8.0621x; 1.4061x over previous
import functools

import jax
import jax.numpy as jnp
from jax import lax
from jax.experimental import pallas as pl
from jax.experimental.pallas import tpu as pltpu
from jax.experimental.pallas import tpu_sc as plsc

_NUM_CORES = 2
_NUM_SUBCORES = 16
_NW = _NUM_CORES * _NUM_SUBCORES
_CHUNK = 128


@functools.partial(jax.jit, static_argnums=(2,))
def _sc_gather(idx2d, table, n_rows_per_w):
    n_total = idx2d.shape[0] * _CHUNK
    d = table.shape[1]
    mesh = plsc.VectorSubcoreMesh(core_axis_name="c", subcore_axis_name="s")

    @functools.partial(
        pl.kernel,
        mesh=mesh,
        out_type=jax.ShapeDtypeStruct((n_total, d), jnp.float32),
        scratch_types=[
            pltpu.VMEM((n_rows_per_w, _CHUNK), jnp.int32),
            pltpu.VMEM((_CHUNK, d), jnp.float32),
            pltpu.VMEM((_CHUNK, d), jnp.float32),
            pltpu.SemaphoreType.DMA,
        ],
        compiler_params=pltpu.CompilerParams(
            use_tc_tiling_on_sc=False, needs_layout_passes=False),
    )
    def k(idx_hbm, table_hbm, out_hbm, idx_v, rows_a, rows_b, gsem):
        wid = lax.axis_index("s") * _NUM_CORES + lax.axis_index("c")
        row0 = wid * n_rows_per_w
        pltpu.sync_copy(idx_hbm.at[pl.ds(row0, n_rows_per_w)], idx_v)

        def out_slice(j):
            return out_hbm.at[pl.ds((row0 + j) * _CHUNK, _CHUNK), :]

        pltpu.async_copy(table_hbm.at[idx_v.at[0]], rows_a, gsem)

        def body(j, carry):
            @pl.when(j % 2 == 0)
            def _even():
                @pl.when(j + 1 < n_rows_per_w)
                def _():
                    pltpu.async_copy(
                        table_hbm.at[idx_v.at[j + 1]], rows_b, gsem)
                pltpu.make_async_copy(
                    table_hbm.at[idx_v.at[j]], rows_a, gsem).wait()
                pltpu.sync_copy(rows_a, out_slice(j))

            @pl.when(j % 2 == 1)
            def _odd():
                @pl.when(j + 1 < n_rows_per_w)
                def _():
                    pltpu.async_copy(
                        table_hbm.at[idx_v.at[j + 1]], rows_a, gsem)
                pltpu.make_async_copy(
                    table_hbm.at[idx_v.at[j]], rows_b, gsem).wait()
                pltpu.sync_copy(rows_b, out_slice(j))
            return carry

        lax.fori_loop(0, n_rows_per_w, body, 0)

    return k(idx2d, table)


def kernel(indices, table):
    b, h = indices.shape
    v, d = table.shape
    n = b * h
    assert n % (_NW * _CHUNK) == 0
    n_rows_per_w = n // (_NW * _CHUNK)
    idx2d = indices.astype(jnp.int32).reshape(n // _CHUNK, _CHUNK)
    t2 = lax.optimization_barrier(table.reshape(v // 2, 2 * d))
    t3 = t2.reshape(v, d)
    out = _sc_gather(idx2d, t3, n_rows_per_w)
    o2 = lax.optimization_barrier(out.reshape(n // 2, 2 * d))
    return o2.reshape(b, h, d)


# speedup vs baseline: 8.0733x; 1.0014x over previous
import functools

import jax
import jax.numpy as jnp
from jax import lax
from jax.experimental import pallas as pl
from jax.experimental.pallas import tpu as pltpu
from jax.experimental.pallas import tpu_sc as plsc

_NUM_CORES = 2
_NUM_SUBCORES = 16
_NW = _NUM_CORES * _NUM_SUBCORES
_CHUNK = 128


@functools.partial(jax.jit, static_argnums=(2,))
def _sc_gather(idx2d, table, n_rows_per_w):
    n_total = idx2d.shape[0] * _CHUNK
    d = table.shape[1]
    mesh = plsc.VectorSubcoreMesh(core_axis_name="c", subcore_axis_name="s")

    @functools.partial(
        pl.kernel,
        mesh=mesh,
        out_type=jax.ShapeDtypeStruct((n_total, d), jnp.float32),
        scratch_types=[
            pltpu.VMEM((n_rows_per_w, _CHUNK), jnp.int32),
            pltpu.VMEM((_CHUNK, d), jnp.float32),
            pltpu.VMEM((_CHUNK, d), jnp.float32),
            pltpu.SemaphoreType.DMA,
        ],
        compiler_params=pltpu.CompilerParams(
            use_tc_tiling_on_sc=False, needs_layout_passes=False),
    )
    def k(idx_hbm, table_hbm, out_hbm, idx_v, rows_a, rows_b, gsem):
        wid = lax.axis_index("s") * _NUM_CORES + lax.axis_index("c")
        row0 = wid * n_rows_per_w
        pltpu.sync_copy(idx_hbm.at[pl.ds(row0, n_rows_per_w)], idx_v)

        def out_slice(j):
            return out_hbm.at[pl.ds((row0 + j) * _CHUNK, _CHUNK), :]

        pltpu.async_copy(table_hbm.at[idx_v.at[0]], rows_a, gsem)

        def body(j, carry):
            @pl.when(j % 2 == 0)
            def _even():
                @pl.when(j + 1 < n_rows_per_w)
                def _():
                    pltpu.async_copy(
                        table_hbm.at[idx_v.at[j + 1]], rows_b, gsem)
                pltpu.make_async_copy(
                    table_hbm.at[idx_v.at[j]], rows_a, gsem).wait()
                pltpu.sync_copy(rows_a, out_slice(j))

            @pl.when(j % 2 == 1)
            def _odd():
                @pl.when(j + 1 < n_rows_per_w)
                def _():
                    pltpu.async_copy(
                        table_hbm.at[idx_v.at[j + 1]], rows_a, gsem)
                pltpu.make_async_copy(
                    table_hbm.at[idx_v.at[j]], rows_b, gsem).wait()
                pltpu.sync_copy(rows_b, out_slice(j))
            return carry

        lax.fori_loop(0, n_rows_per_w, body, 0)

    return k(idx2d, table)


def kernel(indices, table):
    b, h = indices.shape
    v, d = table.shape
    n = b * h
    assert n % (_NW * _CHUNK) == 0
    n_rows_per_w = n // (_NW * _CHUNK)
    idx2d = indices.astype(jnp.int32).reshape(n // _CHUNK, _CHUNK)
    out = _sc_gather(idx2d, table, n_rows_per_w)
    o2 = lax.optimization_barrier(out.reshape(n // 2, 2 * d))
    return o2.reshape(b, h, d)
